# baseline (speedup 1.0000x reference)
import jax
import jax.numpy as jnp
from jax.experimental import pallas as pl
from jax.experimental.pallas import tpu as pltpu

M, N, FN, FE, D, T, NT = 8, 64, 64, 16, 32, 4, 2
MDp = 256
OUTW = 128
ND = N // 2


_F32 = jnp.float32


def _lrelu(v):
    return jnp.where(v > 0, v, 0.01 * v)


def _elu(v):
    return jnp.where(v > 0, v, jnp.exp(jnp.minimum(v, 0.0)) - 1.0)


def _sigmoid(v):
    return 0.5 * (jnp.tanh(0.5 * v) + 1.0)


def _dot(a, b):
    return jnp.dot(a, b, preferred_element_type=_F32)


def _gru(xx, hh, wg, bg):
    gates = _dot(jnp.concatenate([xx, hh], axis=1), wg) + bg
    r = _sigmoid(gates[:, 0:MDp])
    z = _sigmoid(gates[:, MDp:2 * MDp])
    n = jnp.tanh(gates[:, 2 * MDp:3 * MDp] + r * gates[:, 3 * MDp:4 * MDp])
    return (1.0 - z) * n + z * hh


def _softmax_src(lg, mask3):
    lg = jnp.where(mask3 > 0, lg, -1e9)
    mx = jnp.max(lg, axis=1, keepdims=True)
    e = jnp.exp(lg - mx) * mask3
    s = jnp.sum(e, axis=1, keepdims=True)
    return e * pl.reciprocal(jnp.maximum(s, 1e-9), approx=True)


def _iota2(shape, dim):
    return jax.lax.broadcasted_iota(jnp.int32, shape, dim)


def _sel8():
    return (_iota2((MDp, M), 0) == _iota2((MDp, M), 1) * D).astype(_F32)


def _exp8():
    return (_iota2((M, MDp), 1) // D == _iota2((M, MDp), 0)).astype(_F32)


def _stage_a(x_ref, xown_ref, ef_ref, adj_ref, win_ref, w128_ref, w256_ref,
             b_ref, out_ref):
    e_src = (_dot(x_ref[...], win_ref[:, MDp:2 * MDp])
             + b_ref[:, MDp:2 * MDp])
    hv_own = _lrelu(_dot(xown_ref[...], win_ref[:, 0:MDp]) + b_ref[:, 0:MDp])

    ee = _dot(ef_ref[...], win_ref[0:FE, 2 * MDp:3 * MDp])
    he1 = _lrelu(e_src[None, :, :] + ee.reshape(ND, N, MDp))
    he2d = he1.reshape(ND * N, MDp)

    s8 = _sel8()
    wa_e8 = _dot(w128_ref[:, MDp:2 * MDp], s8)
    wa_d8 = _dot(w128_ref[:, 0:MDp], s8)
    ba8 = _dot(b_ref[:, 2 * MDp:3 * MDp], s8)
    att = _dot(he2d, wa_e8).reshape(ND, N, M)
    dst8 = _dot(hv_own, wa_d8)
    lg = _lrelu(dst8[:, None, :] + att + ba8[:, None, :])

    mask3 = adj_ref[...][:, :, None]
    lg = jnp.where(mask3 > 0, lg, -1e9)
    mx = jnp.max(lg, axis=1, keepdims=True)
    e = jnp.exp(lg - mx) * mask3
    ssum = jnp.sum(e, axis=1, keepdims=True)
    rec = pl.reciprocal(jnp.maximum(ssum, 1e-9), approx=True)
    a1c = e * rec
    asum = ssum * rec

    e8 = _exp8()
    a1f = _dot(a1c.reshape(ND * N, M), e8).reshape(ND, N, MDp)
    wh1 = jnp.sum(a1f * he1, axis=1)
    asumf = _dot(asum.reshape(ND, M), e8)
    ctx = _elu(_dot(wh1, w128_ref[:, 2 * MDp:3 * MDp])
               + b_ref[:, 3 * MDp:4 * MDp] * asumf)
    out_ref[...] = jnp.maximum(
        _gru(ctx, hv_own, w256_ref[...], b_ref[:, 4 * MDp:8 * MDp]), 0.0)


def _stage_b(hv_ref, adj_ref, w128_ref, w256_ref, b_ref, out_ref):
    hv = hv_ref[...]
    hv2 = _dot(hv, w128_ref[:, 768:768 + 3 * MDp]) + b_ref[:, 2048:2048
                                                           + 3 * MDp]
    att_dst = hv2[:, 0:MDp]
    att_src = hv2[:, MDp:2 * MDp]
    proj = hv2[:, 2 * MDp:3 * MDp]

    s8 = _sel8()
    ad8 = _dot(att_dst, s8)
    as8 = _dot(att_src, s8)
    ea = (_iota2((M, M * N), 1) // N == _iota2((M, M * N), 0)).astype(_F32)
    ad_l = _dot(ad8, ea)
    d64 = (_iota2((N, M * N), 0)
           == (_iota2((N, M * N), 1) & (N - 1))).astype(_F32)
    as_l = jnp.sum(_dot(as8, ea) * d64, axis=0, keepdims=True)
    adjt = jnp.concatenate([adj_ref[...]] * M, axis=1)

    lg = _lrelu(ad_l + as_l)
    lg = jnp.where(adjt > 0, lg, -1e9).reshape(N, M, N)
    mx = jnp.max(lg, axis=2, keepdims=True)
    e = jnp.exp(lg - mx) * adjt.reshape(N, M, N)
    ssum = jnp.sum(e, axis=2, keepdims=True)
    a2c = (e * pl.reciprocal(jnp.maximum(ssum, 1e-9), approx=True)
           ).reshape(N, M * N)

    pmask = (_iota2((M * N, MDp), 0) // N
             == _iota2((M * N, MDp), 1) // D).astype(_F32)
    p_full = jnp.broadcast_to(proj[None, :, :], (M, N, MDp)).reshape(M * N,
                                                                     MDp)
    ctx2 = _elu(_dot(a2c, p_full * pmask))
    hv = jnp.maximum(
        _gru(ctx2, hv, w256_ref[:, 4 * MDp:8 * MDp],
             b_ref[:, 2816:2816 + 4 * MDp]),
        0.0)

    g = jnp.sum(hv, axis=0, keepdims=True)
    for t in range(T):
        hv3t = (_dot(hv, w128_ref[:, 1536 + 2 * MDp * t:1536
                                    + 2 * MDp * (t + 1)])
                + b_ref[:, 3840 + 2 * MDp * t:3840 + 2 * MDp * (t + 1)])
        hv_att = hv3t[:, 0:MDp]
        pn = hv3t[:, MDp:2 * MDp]
        zt = _lrelu(_dot(jnp.maximum(g, 0.0),
                         w128_ref[:, 3584 + MDp * t:3584 + MDp * (t + 1)])
                    + hv_att)
        zt = zt - jnp.max(zt, axis=0, keepdims=True)
        a = jnp.exp(zt)
        a = a * pl.reciprocal(jnp.maximum(jnp.sum(a, axis=0, keepdims=True),
                                          1e-9), approx=True)
        rctx = _elu(jnp.sum(a * pn, axis=0, keepdims=True))
        g = jnp.maximum(
            _gru(rctx, g,
                 w256_ref[:, 2048 + 4 * MDp * t:2048 + 4 * MDp * (t + 1)],
                 b_ref[:, 5888 + 4 * MDp * t:5888 + 4 * MDp * (t + 1)]),
            0.0)

    out_ref[...] = (_dot(g, w128_ref[:, 4608:4608 + OUTW])
                    + b_ref[:, 9984:9984 + OUTW])


def kernel(node_feats, edge_feats, adj, W_in, W128, W256, B):
    ef2 = edge_feats.reshape(N * N, FE)
    b_cols = B.shape[1]

    hv1 = pl.pallas_call(
        _stage_a,
        out_shape=jax.ShapeDtypeStruct((N, MDp), _F32),
        grid=(2,),
        in_specs=[
            pl.BlockSpec((N, FN), lambda i: (0, 0)),
            pl.BlockSpec((ND, FN), lambda i: (i, 0)),
            pl.BlockSpec((ND * N, FE), lambda i: (i, 0)),
            pl.BlockSpec((ND, N), lambda i: (i, 0)),
            pl.BlockSpec((FN, 3 * MDp), lambda i: (0, 0)),
            pl.BlockSpec((MDp, 3 * MDp), lambda i: (0, 0)),
            pl.BlockSpec((2 * MDp, 4 * MDp), lambda i: (0, 0)),
            pl.BlockSpec((1, b_cols), lambda i: (0, 0)),
        ],
        out_specs=pl.BlockSpec((ND, MDp), lambda i: (i, 0)),
        compiler_params=pltpu.CompilerParams(
            dimension_semantics=("parallel",)),
    )(node_feats, node_feats, ef2, adj, W_in, W128, W256, B)

    out = pl.pallas_call(
        _stage_b,
        out_shape=jax.ShapeDtypeStruct((1, OUTW), _F32),
        grid=(1,),
        in_specs=[
            pl.BlockSpec((N, MDp), lambda i: (0, 0)),
            pl.BlockSpec((N, N), lambda i: (0, 0)),
            pl.BlockSpec((MDp, 4736), lambda i: (0, 0)),
            pl.BlockSpec((2 * MDp, 6144), lambda i: (0, 0)),
            pl.BlockSpec((1, b_cols), lambda i: (0, 0)),
        ],
        out_specs=pl.BlockSpec((1, OUTW), lambda i: (0, 0)),
        compiler_params=pltpu.CompilerParams(
            dimension_semantics=("arbitrary",)),
    )(hv1, adj, W128, W256, B)

    return out[0, :M * NT].reshape(M, 1, NT)


# speedup vs baseline: 1.0353x; 1.0353x over previous
import jax
import jax.numpy as jnp
from jax.experimental import pallas as pl
from jax.experimental.pallas import tpu as pltpu

M, N, FN, FE, D, T, NT = 8, 64, 64, 16, 32, 4, 2
MDp = 256
OUTW = 128
ND = N // 2


_F32 = jnp.float32


def _lrelu(v):
    return jnp.where(v > 0, v, 0.01 * v)


def _elu(v):
    return jnp.where(v > 0, v, jnp.exp(jnp.minimum(v, 0.0)) - 1.0)


def _sigmoid(v):
    return 0.5 * (jnp.tanh(0.5 * v) + 1.0)


def _dot(a, b):
    return jnp.dot(a, b, preferred_element_type=_F32)


def _gru(xx, hh, wg, bg):
    gates = _dot(jnp.concatenate([xx, hh], axis=1), wg) + bg
    r = _sigmoid(gates[:, 0:MDp])
    z = _sigmoid(gates[:, MDp:2 * MDp])
    n = jnp.tanh(gates[:, 2 * MDp:3 * MDp] + r * gates[:, 3 * MDp:4 * MDp])
    return (1.0 - z) * n + z * hh


def _softmax_src(lg, mask3):
    lg = jnp.where(mask3 > 0, lg, -1e9)
    mx = jnp.max(lg, axis=1, keepdims=True)
    e = jnp.exp(lg - mx) * mask3
    s = jnp.sum(e, axis=1, keepdims=True)
    return e * pl.reciprocal(jnp.maximum(s, 1e-9), approx=True)


def _iota2(shape, dim):
    return jax.lax.broadcasted_iota(jnp.int32, shape, dim)


def _sel8():
    return (_iota2((MDp, M), 0) == _iota2((MDp, M), 1) * D).astype(_F32)


def _exp8():
    return (_iota2((M, MDp), 1) // D == _iota2((M, MDp), 0)).astype(_F32)


def _stage_a(x_ref, xown_ref, ef_ref, adj_ref, win_ref, w128_ref, w256_ref,
             b_ref, out_ref):
    e_src = (_dot(x_ref[...], win_ref[:, MDp:2 * MDp])
             + b_ref[:, MDp:2 * MDp])
    hv_own = _lrelu(_dot(xown_ref[...], win_ref[:, 0:MDp]) + b_ref[:, 0:MDp])

    ee = _dot(ef_ref[...], win_ref[0:FE, 2 * MDp:3 * MDp])
    he1 = _lrelu(e_src[None, :, :] + ee.reshape(ND, N, MDp))
    he2d = he1.reshape(ND * N, MDp)

    s8 = _sel8()
    wa_e8 = _dot(w128_ref[:, MDp:2 * MDp], s8)
    wa_d8 = _dot(w128_ref[:, 0:MDp], s8)
    ba8 = _dot(b_ref[:, 2 * MDp:3 * MDp], s8)
    att = _dot(he2d, wa_e8).reshape(ND, N, M)
    dst8 = _dot(hv_own, wa_d8)
    lg = _lrelu(dst8[:, None, :] + att + ba8[:, None, :])

    mask3 = adj_ref[...][:, :, None]
    lg = jnp.where(mask3 > 0, lg, -1e9)
    mx = jnp.max(lg, axis=1, keepdims=True)
    e = jnp.exp(lg - mx) * mask3
    ssum = jnp.sum(e, axis=1, keepdims=True)
    rec = pl.reciprocal(jnp.maximum(ssum, 1e-9), approx=True)
    a1c = e * rec
    asum = ssum * rec

    e8 = _exp8()
    a1f = _dot(a1c.reshape(ND * N, M), e8).reshape(ND, N, MDp)
    wh1 = jnp.sum(a1f * he1, axis=1)
    asumf = _dot(asum.reshape(ND, M), e8)
    ctx = _elu(_dot(wh1, w128_ref[:, 2 * MDp:3 * MDp])
               + b_ref[:, 3 * MDp:4 * MDp] * asumf)
    out_ref[...] = jnp.maximum(
        _gru(ctx, hv_own, w256_ref[...], b_ref[:, 4 * MDp:8 * MDp]), 0.0)


def _stage_b(hv_ref, adj_ref, w128b_ref, whv3_0, whv3_1, whv3_2, whv3_3,
             wrg_01, wrg_23, wp_ref, w256b_ref, wgrur_01, wgrur_23, b_ref,
             out_ref):
    hv = hv_ref[...]
    hv2 = _dot(hv, w128b_ref[...]) + b_ref[:, 2048:2048 + 3 * MDp]
    att_dst = hv2[:, 0:MDp]
    att_src = hv2[:, MDp:2 * MDp]
    proj = hv2[:, 2 * MDp:3 * MDp]

    s8 = _sel8()
    ad8 = _dot(att_dst, s8)
    as8 = _dot(att_src, s8)
    ea = (_iota2((M, M * N), 1) // N == _iota2((M, M * N), 0)).astype(_F32)
    ad_l = _dot(ad8, ea)
    d64 = (_iota2((N, M * N), 0)
           == (_iota2((N, M * N), 1) & (N - 1))).astype(_F32)
    as_l = jnp.sum(_dot(as8, ea) * d64, axis=0, keepdims=True)
    adjt = jnp.concatenate([adj_ref[...]] * M, axis=1)

    lg = _lrelu(ad_l + as_l)
    lg = jnp.where(adjt > 0, lg, -1e9).reshape(N, M, N)
    mx = jnp.max(lg, axis=2, keepdims=True)
    e = jnp.exp(lg - mx) * adjt.reshape(N, M, N)
    ssum = jnp.sum(e, axis=2, keepdims=True)
    a2c = (e * pl.reciprocal(jnp.maximum(ssum, 1e-9), approx=True)
           ).reshape(N, M * N)

    pmask = (_iota2((M * N, MDp), 0) // N
             == _iota2((M * N, MDp), 1) // D).astype(_F32)
    p_full = jnp.broadcast_to(proj[None, :, :], (M, N, MDp)).reshape(M * N,
                                                                     MDp)
    ctx2 = _elu(_dot(a2c, p_full * pmask))
    hv = jnp.maximum(
        _gru(ctx2, hv, w256b_ref[:, 0:4 * MDp], b_ref[:, 2816:2816 + 4 * MDp]),
        0.0)

    whv3 = (whv3_0, whv3_1, whv3_2, whv3_3)
    wrg = (wrg_01, wrg_23)
    wgrur = (wgrur_01, wgrur_23)
    g = jnp.sum(hv, axis=0, keepdims=True)
    for t in range(T):
        hv3t = (_dot(hv, whv3[t][...])
                + b_ref[:, 3840 + 2 * MDp * t:3840 + 2 * MDp * (t + 1)])
        hv_att = hv3t[:, 0:MDp]
        pn = hv3t[:, MDp:2 * MDp]
        zt = _lrelu(_dot(jnp.maximum(g, 0.0),
                         wrg[t // 2][:, (t % 2) * MDp:(t % 2 + 1) * MDp])
                    + hv_att)
        zt = zt - jnp.max(zt, axis=0, keepdims=True)
        a = jnp.exp(zt)
        a = a * pl.reciprocal(jnp.maximum(jnp.sum(a, axis=0, keepdims=True),
                                          1e-9), approx=True)
        rctx = _elu(jnp.sum(a * pn, axis=0, keepdims=True))
        g = jnp.maximum(
            _gru(rctx, g,
                 wgrur[t // 2][:, (t % 2) * 4 * MDp:(t % 2 + 1) * 4 * MDp],
                 b_ref[:, 5888 + 4 * MDp * t:5888 + 4 * MDp * (t + 1)]),
            0.0)

    out_ref[...] = _dot(g, wp_ref[...]) + b_ref[:, 9984:9984 + OUTW]


def kernel(node_feats, edge_feats, adj, W_in, W128, W256, B):
    ef2 = edge_feats.reshape(N * N, FE)
    b_cols = B.shape[1]

    hv1 = pl.pallas_call(
        _stage_a,
        out_shape=jax.ShapeDtypeStruct((N, MDp), _F32),
        grid=(2,),
        in_specs=[
            pl.BlockSpec((N, FN), lambda i: (0, 0)),
            pl.BlockSpec((ND, FN), lambda i: (i, 0)),
            pl.BlockSpec((ND * N, FE), lambda i: (i, 0)),
            pl.BlockSpec((ND, N), lambda i: (i, 0)),
            pl.BlockSpec((FN, 3 * MDp), lambda i: (0, 0)),
            pl.BlockSpec((MDp, 3 * MDp), lambda i: (0, 0)),
            pl.BlockSpec((2 * MDp, 4 * MDp), lambda i: (0, 0)),
            pl.BlockSpec((1, b_cols), lambda i: (0, 0)),
        ],
        out_specs=pl.BlockSpec((ND, MDp), lambda i: (i, 0)),
        compiler_params=pltpu.CompilerParams(
            dimension_semantics=("parallel",)),
    )(node_feats, node_feats, ef2, adj, W_in, W128, W256, B)

    out = pl.pallas_call(
        _stage_b,
        out_shape=jax.ShapeDtypeStruct((1, OUTW), _F32),
        grid=(1,),
        in_specs=[
            pl.BlockSpec((N, MDp), lambda i: (0, 0)),
            pl.BlockSpec((N, N), lambda i: (0, 0)),
            pl.BlockSpec((MDp, 3 * MDp), lambda i: (0, 1)),
            pl.BlockSpec((MDp, 512), lambda i: (0, 3)),
            pl.BlockSpec((MDp, 512), lambda i: (0, 4)),
            pl.BlockSpec((MDp, 512), lambda i: (0, 5)),
            pl.BlockSpec((MDp, 512), lambda i: (0, 6)),
            pl.BlockSpec((MDp, 512), lambda i: (0, 7)),
            pl.BlockSpec((MDp, 512), lambda i: (0, 8)),
            pl.BlockSpec((MDp, OUTW), lambda i: (0, 36)),
            pl.BlockSpec((2 * MDp, 4 * MDp), lambda i: (0, 1)),
            pl.BlockSpec((2 * MDp, 8 * MDp), lambda i: (0, 1)),
            pl.BlockSpec((2 * MDp, 8 * MDp), lambda i: (0, 2)),
            pl.BlockSpec((1, b_cols), lambda i: (0, 0)),
        ],
        out_specs=pl.BlockSpec((1, OUTW), lambda i: (0, 0)),
        compiler_params=pltpu.CompilerParams(
            dimension_semantics=("arbitrary",)),
    )(hv1, adj, W128, W128, W128, W128, W128, W128, W128, W128,
      W256, W256, W256, B)

    return out[0, :M * NT].reshape(M, 1, NT)
